# baseline (speedup 1.0000x reference)
import functools

import jax
import jax.numpy as jnp
from jax.experimental import pallas as pl
from jax.experimental.pallas import tpu as pltpu


def _eca_body(w_ref, x_ref, o_ref, *, k, pad, c, inv_hw):
    x = x_ref[...]
    s = jnp.sum(x.astype(jnp.float32), axis=0)

    if pad:
        z = jnp.zeros((s.shape[0], pad), dtype=jnp.float32)
        sp = jnp.concatenate([z, s, z], axis=-1)
    else:
        sp = s
    acc = (w_ref[0] * inv_hw) * sp[:, 0:c]
    for j in range(1, k):
        acc = acc + (w_ref[j] * inv_hw) * sp[:, j:j + c]

    gate = jax.nn.sigmoid(acc).astype(x.dtype)
    o_ref[...] = x * gate[None, :, :]


def kernel(x, conv_w):
    B, C, H, W = x.shape
    HW = H * W
    k = int(conv_w.shape[0])
    pad = k // 2

    xt = jnp.transpose(x.reshape(B, C, HW), (2, 0, 1))

    bt = 32
    while B % bt:
        bt //= 2
    bt = max(bt, 8) if B % 8 == 0 else B
    grid = (B // bt,)

    body = functools.partial(_eca_body, k=k, pad=pad, c=C, inv_hw=1.0 / HW)
    out = pl.pallas_call(
        body,
        out_shape=jax.ShapeDtypeStruct((HW, B, C), x.dtype),
        grid_spec=pltpu.PrefetchScalarGridSpec(
            num_scalar_prefetch=0,
            grid=grid,
            in_specs=[
                pl.BlockSpec(memory_space=pltpu.MemorySpace.SMEM),
                pl.BlockSpec((HW, bt, C), lambda i: (0, i, 0)),
            ],
            out_specs=pl.BlockSpec((HW, bt, C), lambda i: (0, i, 0)),
        ),
        compiler_params=pltpu.CompilerParams(
            dimension_semantics=("parallel",),
            vmem_limit_bytes=64 * 1024 * 1024,
        ),
    )(conv_w.astype(jnp.float32), xt)
    return jnp.transpose(out, (1, 2, 0)).reshape(B, C, H, W)
